# baseline (speedup 1.0000x reference)
import functools

import jax
import jax.numpy as jnp
from jax.experimental import pallas as pl
from jax.experimental.pallas import tpu as pltpu

D = 100000
B = 128
DBLK = 4096
NB = (D + DBLK - 1) // DBLK


def _body(prob_ref, m_ref, ls_ref, u_ref, eps_ref,
          out_ref, y_ref,
          best_val, best_idx, best_prob, m_lse, s_lse):
    i = pl.program_id(0)
    colbase = i * DBLK

    lane = jax.lax.broadcasted_iota(jnp.int32, (1, DBLK), 1)
    valid = (colbase + lane) < D

    pb = prob_ref[...]
    u = u_ref[...]
    e = -jnp.log(u)
    c = jnp.exp(pb)
    val = jnp.where(valid, c / e, -1.0)

    out_ref[...] = m_ref[...] + jnp.exp(ls_ref[...]) * eps_ref[...]

    @pl.when(i == 0)
    def _init():
        best_val[...] = jnp.full((B, 1), -1.0, jnp.float32)
        best_idx[...] = jnp.zeros((B, 1), jnp.int32)
        best_prob[...] = jnp.zeros((B, 1), jnp.float32)
        m_lse[...] = jnp.full((1, 1), -jnp.inf, jnp.float32)
        s_lse[...] = jnp.zeros((1, 1), jnp.float32)

    local_max = jnp.max(val, axis=1, keepdims=True)
    w = val == local_max
    local_arg = jnp.min(jnp.where(w, lane, jnp.iinfo(jnp.int32).max),
                        axis=1, keepdims=True)
    local_prob = jnp.max(jnp.where(w, pb, -jnp.inf),
                         axis=1, keepdims=True)

    upd = local_max > best_val[...]
    best_val[...] = jnp.where(upd, local_max, best_val[...])
    best_idx[...] = jnp.where(upd, colbase + local_arg, best_idx[...])
    best_prob[...] = jnp.where(upd, local_prob, best_prob[...])

    pbm = jnp.where(valid, pb, -jnp.inf)
    bm = jnp.max(pbm, keepdims=True).reshape(1, 1)
    m_old = m_lse[...]
    m_new = jnp.maximum(m_old, bm)
    s_lse[...] = (s_lse[...] * jnp.exp(m_old - m_new)
                  + jnp.sum(jnp.exp(pbm - m_new), keepdims=True).reshape(1, 1))
    m_lse[...] = m_new

    @pl.when(i == NB - 1)
    def _finalize():
        lse = m_lse[...] + jnp.log(s_lse[...])
        y_ref[...] = best_idx[...]
        off = D - (NB - 1) * DBLK
        out_ref[:, off:off + 1] = best_prob[...] - lse


@jax.jit
def kernel(prob, m_z, log_s_z, u, eps):
    prob2 = prob.reshape(1, D)
    m2 = m_z.reshape(1, D)
    ls2 = log_s_z.reshape(1, D)

    row_spec = pl.BlockSpec((1, DBLK), lambda i: (0, i))
    mat_spec = pl.BlockSpec((B, DBLK), lambda i: (0, i))

    out, y2 = pl.pallas_call(
        _body,
        grid=(NB,),
        in_specs=[row_spec, row_spec, row_spec, mat_spec, mat_spec],
        out_specs=[
            pl.BlockSpec((B, DBLK), lambda i: (0, i)),
            pl.BlockSpec((B, 1), lambda i: (0, 0)),
        ],
        out_shape=[
            jax.ShapeDtypeStruct((B, D + 1), jnp.float32),
            jax.ShapeDtypeStruct((B, 1), jnp.int32),
        ],
        scratch_shapes=[
            pltpu.VMEM((B, 1), jnp.float32),
            pltpu.VMEM((B, 1), jnp.int32),
            pltpu.VMEM((B, 1), jnp.float32),
            pltpu.VMEM((1, 1), jnp.float32),
            pltpu.VMEM((1, 1), jnp.float32),
        ],
    )(prob2, m2, ls2, u, eps)
    return (y2.reshape(B), out)


# speedup vs baseline: 1.0160x; 1.0160x over previous
import jax
import jax.numpy as jnp
from jax.experimental import pallas as pl
from jax.experimental.pallas import tpu as pltpu

D = 100000
B = 128
DBLK = 4096
NB = (D + DBLK - 1) // DBLK
_I32MAX = jnp.iinfo(jnp.int32).max


def _body(prob_ref, m_ref, ls_ref, u_ref, eps_ref, pfull_ref,
          out_ref, y_ref,
          best_key, best_idx, best_prob):
    i = pl.program_id(0)
    lane = jax.lax.broadcasted_iota(jnp.int32, (1, DBLK), 1)

    pb = prob_ref[...]
    ic = jnp.exp(-pb)
    e = -jnp.log(u_ref[...])
    key_raw = e * ic

    out_ref[...] = m_ref[...] + jnp.exp(ls_ref[...]) * eps_ref[...]

    @pl.when(i == 0)
    def _init():
        best_key[...] = jnp.full((B, 1), jnp.inf, jnp.float32)
        best_idx[...] = jnp.zeros((B, 1), jnp.int32)
        best_prob[...] = jnp.zeros((B, 1), jnp.float32)

    def update(key):
        local_min = jnp.min(key, axis=1, keepdims=True)
        w = key == local_min
        local_arg = jnp.min(jnp.where(w, lane, _I32MAX),
                            axis=1, keepdims=True)
        local_prob = jnp.max(jnp.where(w, pb, -jnp.inf),
                             axis=1, keepdims=True)
        upd = local_min < best_key[...]
        best_key[...] = jnp.where(upd, local_min, best_key[...])
        best_idx[...] = jnp.where(upd, i * DBLK + local_arg, best_idx[...])
        best_prob[...] = jnp.where(upd, local_prob, best_prob[...])

    @pl.when(i < NB - 1)
    def _main():
        update(key_raw)

    @pl.when(i == NB - 1)
    def _last():
        update(jnp.where(lane < (D - i * DBLK), key_raw, jnp.inf))

        pf = pfull_ref[...]
        mx = jnp.max(pf, keepdims=True).reshape(1, 1)
        s = jnp.sum(jnp.exp(pf - mx), keepdims=True).reshape(1, 1)
        lse = mx + jnp.log(s)
        y_ref[...] = best_idx[...]
        off = D - (NB - 1) * DBLK
        out_ref[:, off:off + 1] = best_prob[...] - lse


@jax.jit
def kernel(prob, m_z, log_s_z, u, eps):
    prob2 = prob.reshape(1, D)
    m2 = m_z.reshape(1, D)
    ls2 = log_s_z.reshape(1, D)

    row_spec = pl.BlockSpec((1, DBLK), lambda i: (0, i))
    mat_spec = pl.BlockSpec((B, DBLK), lambda i: (0, i))
    full_spec = pl.BlockSpec((1, D), lambda i: (0, 0))

    out, y2 = pl.pallas_call(
        _body,
        grid=(NB,),
        in_specs=[row_spec, row_spec, row_spec, mat_spec, mat_spec,
                  full_spec],
        out_specs=[
            pl.BlockSpec((B, DBLK), lambda i: (0, i)),
            pl.BlockSpec((B, 1), lambda i: (0, 0)),
        ],
        out_shape=[
            jax.ShapeDtypeStruct((B, D + 1), jnp.float32),
            jax.ShapeDtypeStruct((B, 1), jnp.int32),
        ],
        scratch_shapes=[
            pltpu.VMEM((B, 1), jnp.float32),
            pltpu.VMEM((B, 1), jnp.int32),
            pltpu.VMEM((B, 1), jnp.float32),
        ],
    )(prob2, m2, ls2, u, eps, prob2)
    return (y2.reshape(B), out)


# speedup vs baseline: 1.0630x; 1.0462x over previous
import jax
import jax.numpy as jnp
from jax.experimental import pallas as pl
from jax.experimental.pallas import tpu as pltpu

D = 100000
B = 128
CW = 4096
NCH = (D + CW - 1) // CW
LAST = D - (NCH - 1) * CW
NBUF = 4
LA = NBUF - 1
_I32MAX = jnp.iinfo(jnp.int32).max


def _in_copies(u_hbm, e_hbm, ub, eb, su, se, j):
    slot = jax.lax.rem(j, NBUF)
    cu = pltpu.make_async_copy(
        u_hbm.at[:, pl.ds(j * CW, CW)], ub.at[slot], su.at[slot])
    ce = pltpu.make_async_copy(
        e_hbm.at[:, pl.ds(j * CW, CW)], eb.at[slot], se.at[slot])
    return cu, ce


def _in_copies_last(u_hbm, e_hbm, ubl, ebl, sul, sel):
    base = (NCH - 1) * CW
    cu = pltpu.make_async_copy(u_hbm.at[:, pl.ds(base, LAST)], ubl, sul)
    ce = pltpu.make_async_copy(e_hbm.at[:, pl.ds(base, LAST)], ebl, sel)
    return cu, ce


def _out_copy(out_hbm, ob, so, j):
    slot = jax.lax.rem(j, NBUF)
    return pltpu.make_async_copy(
        ob.at[slot], out_hbm.at[:, pl.ds(j * CW, CW)], so.at[slot])


def _out_copy_last(out_hbm, obl, sol):
    base = (NCH - 1) * CW
    return pltpu.make_async_copy(
        obl, out_hbm.at[:, pl.ds(base, LAST + 1)], sol)


def _body(pb_ref, m_ref, ls_ref, pfull_ref, u_hbm, e_hbm,
          out_hbm, y_ref,
          ub, eb, ob, ubl, ebl, obl,
          su, se, so, sul, sel, sol,
          bk, bi, bp):
    i = pl.program_id(0)
    slot = jax.lax.rem(i, NBUF)

    @pl.when(i == 0)
    def _prologue():
        bk[...] = jnp.full((B, 1), jnp.inf, jnp.float32)
        bi[...] = jnp.zeros((B, 1), jnp.int32)
        bp[...] = jnp.zeros((B, 1), jnp.float32)
        for j in range(LA):
            cu, ce = _in_copies(u_hbm, e_hbm, ub, eb, su, se, j)
            cu.start()
            ce.start()

    j = i + LA

    @pl.when(j < NCH - 1)
    def _start_full():
        cu, ce = _in_copies(u_hbm, e_hbm, ub, eb, su, se, j)
        cu.start()
        ce.start()

    @pl.when(j == NCH - 1)
    def _start_last():
        cu, ce = _in_copies_last(u_hbm, e_hbm, ubl, ebl, sul, sel)
        cu.start()
        ce.start()

    @pl.when(jnp.logical_and(i >= NBUF, i < NCH - 1))
    def _drain_out():
        _out_copy(out_hbm, ob, so, i - NBUF).wait()

    def update(key, pb_c, lane):
        local_min = jnp.min(key, axis=1, keepdims=True)
        w = key == local_min
        local_arg = jnp.min(jnp.where(w, lane, _I32MAX),
                            axis=1, keepdims=True)
        local_prob = jnp.max(jnp.where(w, pb_c, -jnp.inf),
                             axis=1, keepdims=True)
        upd = local_min < bk[...]
        bk[...] = jnp.where(upd, local_min, bk[...])
        bi[...] = jnp.where(upd, i * CW + local_arg, bi[...])
        bp[...] = jnp.where(upd, local_prob, bp[...])

    @pl.when(i < NCH - 1)
    def _compute_full():
        cu, ce = _in_copies(u_hbm, e_hbm, ub, eb, su, se, i)
        cu.wait()
        ce.wait()
        pb = pb_ref[...]
        lane = jax.lax.broadcasted_iota(jnp.int32, (1, CW), 1)
        e = -jnp.log(ub[slot])
        key = e * jnp.exp(-pb)
        ob[slot] = m_ref[...] + jnp.exp(ls_ref[...]) * eb[slot]
        update(key, pb, lane)
        _out_copy(out_hbm, ob, so, i).start()

    @pl.when(i == NCH - 1)
    def _compute_last():
        cu, ce = _in_copies_last(u_hbm, e_hbm, ubl, ebl, sul, sel)
        cu.wait()
        ce.wait()
        pb = pb_ref[...][:, :LAST]
        lane = jax.lax.broadcasted_iota(jnp.int32, (1, LAST), 1)
        e = -jnp.log(ubl[...])
        key = e * jnp.exp(-pb)
        obl[:, :LAST] = (m_ref[...][:, :LAST]
                         + jnp.exp(ls_ref[...][:, :LAST]) * ebl[...])
        update(key, pb, lane)

        pf = pfull_ref[...]
        mx = jnp.max(pf, keepdims=True).reshape(1, 1)
        s = jnp.sum(jnp.exp(pf - mx), keepdims=True).reshape(1, 1)
        lse = mx + jnp.log(s)
        y_ref[...] = bi[...]
        obl[:, LAST:LAST + 1] = bp[...] - lse
        _out_copy_last(out_hbm, obl, sol).start()

        for k in range(NCH - 1 - NBUF, NCH - 1):
            _out_copy(out_hbm, ob, so, k).wait()
        _out_copy_last(out_hbm, obl, sol).wait()


@jax.jit
def kernel(prob, m_z, log_s_z, u, eps):
    prob2 = prob.reshape(1, D)
    m2 = m_z.reshape(1, D)
    ls2 = log_s_z.reshape(1, D)

    row_spec = pl.BlockSpec((1, CW), lambda i: (0, i))
    full_spec = pl.BlockSpec((1, D), lambda i: (0, 0))
    any_spec = pl.BlockSpec(memory_space=pltpu.MemorySpace.HBM)

    out, y2 = pl.pallas_call(
        _body,
        grid=(NCH,),
        in_specs=[row_spec, row_spec, row_spec, full_spec,
                  any_spec, any_spec],
        out_specs=[
            any_spec,
            pl.BlockSpec((B, 1), lambda i: (0, 0)),
        ],
        out_shape=[
            jax.ShapeDtypeStruct((B, D + 1), jnp.float32),
            jax.ShapeDtypeStruct((B, 1), jnp.int32),
        ],
        scratch_shapes=[
            pltpu.VMEM((NBUF, B, CW), jnp.float32),
            pltpu.VMEM((NBUF, B, CW), jnp.float32),
            pltpu.VMEM((NBUF, B, CW), jnp.float32),
            pltpu.VMEM((B, LAST), jnp.float32),
            pltpu.VMEM((B, LAST), jnp.float32),
            pltpu.VMEM((B, LAST + 1), jnp.float32),
            pltpu.SemaphoreType.DMA((NBUF,)),
            pltpu.SemaphoreType.DMA((NBUF,)),
            pltpu.SemaphoreType.DMA((NBUF,)),
            pltpu.SemaphoreType.DMA,
            pltpu.SemaphoreType.DMA,
            pltpu.SemaphoreType.DMA,
            pltpu.VMEM((B, 1), jnp.float32),
            pltpu.VMEM((B, 1), jnp.int32),
            pltpu.VMEM((B, 1), jnp.float32),
        ],
    )(prob2, m2, ls2, prob2, u, eps)
    return (y2.reshape(B), out)
